# baseline (speedup 1.0000x reference)
import jax
import jax.numpy as jnp
import numpy as np
from jax.experimental import pallas as pl
from jax.experimental.pallas import tpu as pltpu

S, HID, HEADS = 2048, 1024, 16
CHUNK, TOPK = 64, 4
DH = HID // HEADS
C = S // CHUNK
NQKV = 3 * HID
NCOL = NQKV + 128
BM = 512
BQ = 256
BK = 256
NEG = -1e9


def _proj_kernel(x_ref, w_ref, b_ref, y_ref, km_ref):
    acc = jnp.dot(x_ref[...], w_ref[...], preferred_element_type=jnp.float32)
    acc = acc + b_ref[...]
    y_ref[...] = acc
    r = jax.lax.broadcasted_iota(jnp.int32, (BM // CHUNK, BM), 1)
    c = jax.lax.broadcasted_iota(jnp.int32, (BM // CHUNK, BM), 0)
    m = jnp.where(r // CHUNK == c, 1.0 / CHUNK, 0.0)
    km_ref[...] = jnp.dot(m, acc[:, HID:2 * HID],
                          preferred_element_type=jnp.float32)


def _kk_kernel(ns_ref, kk_ref):
    ns = ns_ref[...]
    col0 = jax.lax.broadcasted_iota(jnp.int32, (S, 128), 1) == 0
    mean = jnp.sum(jnp.where(col0, ns, 0.0)) / S
    cnt = jnp.sum(jnp.where(col0 & (ns > mean), 1, 0))
    cnt = jnp.clip(cnt, 2, S)
    dyn = jnp.floor(jnp.float32(0.8 * float(TOPK))
                    + jnp.float32(0.2) * cnt.astype(jnp.float32))
    kk_ref[0, 0] = jnp.clip(dyn.astype(jnp.int32), 1, C)


def _flash_kernel(kk_ref, q_ref, k_ref, v_ref, km_ref, o_ref):
    i = pl.program_id(1)
    kk = kk_ref[0, 0]
    scale = 1.0 / np.sqrt(DH)
    row = i * BQ + jax.lax.broadcasted_iota(jnp.int32, (BQ, C), 0)
    cid = jax.lax.broadcasted_iota(jnp.int32, (BQ, C), 1)
    qc = row // CHUNK
    qr = i * BQ + jax.lax.broadcasted_iota(jnp.int32, (BQ, BK), 0)
    tE_c = jax.lax.broadcasted_iota(jnp.int32, (C, BK), 0)
    tE_t = jax.lax.broadcasted_iota(jnp.int32, (C, BK), 1) // CHUNK
    kt_loc = jax.lax.broadcasted_iota(jnp.int32, (BQ, BK), 1)

    outs = []
    for t in range(2):
        q = q_ref[:, t * DH:(t + 1) * DH]
        km = km_ref[:, t * DH:(t + 1) * DH]

        g = jax.lax.dot_general(q, km, (((1,), (1,)), ((), ())),
                                preferred_element_type=jnp.float32)
        g = jnp.where(cid > qc, NEG, g)
        g = jnp.where(cid == qc, 1e9, g)

        rank = jnp.zeros((BQ, C), jnp.int32)
        for j in range(C):
            gj = g[:, j:j + 1]
            beat = (gj > g) | ((gj == g) & (j < cid))
            rank = rank + beat.astype(jnp.int32)
        sel = ((rank < kk) & (g > -1e8)).astype(jnp.float32)

        def body(jb, carry):
            m, l, acc = carry
            kb = k_ref[pl.ds(jb * BK, BK), t * DH:(t + 1) * DH]
            vb = v_ref[pl.ds(jb * BK, BK), t * DH:(t + 1) * DH]
            s = jax.lax.dot_general(q, kb, (((1,), (1,)), ((), ())),
                                    preferred_element_type=jnp.float32)
            s = s * scale
            e = jnp.where(tE_c == jb * (BK // CHUNK) + tE_t, 1.0, 0.0)
            tok = jnp.dot(sel, e, preferred_element_type=jnp.float32)
            ok = (tok > 0.5) & (jb * BK + kt_loc <= qr)
            s = jnp.where(ok, s, NEG)
            mn = jnp.maximum(m, jnp.max(s, axis=1, keepdims=True))
            p = jnp.exp(s - mn)
            alpha = jnp.exp(m - mn)
            l2 = l * alpha + jnp.sum(p, axis=1, keepdims=True)
            acc2 = acc * alpha + jnp.dot(p, vb,
                                         preferred_element_type=jnp.float32)
            return mn, l2, acc2

        m0 = jnp.full((BQ, 1), -1e30, jnp.float32)
        l0 = jnp.zeros((BQ, 1), jnp.float32)
        a0 = jnp.zeros((BQ, DH), jnp.float32)
        m, l, acc = jax.lax.fori_loop(0, i + 1, body, (m0, l0, a0))
        outs.append(acc / l)
    o_ref[...] = jnp.concatenate(outs, axis=1)


def _layernorm(x, g, b):
    mu = jnp.mean(x, axis=1, keepdims=True)
    var = jnp.mean((x - mu) ** 2, axis=1, keepdims=True)
    return (x - mu) / jnp.sqrt(var + 1e-6) * g + b


def _mlp_kernel(x_ref, w1_ref, b1_ref, geln_ref, beln_ref,
                w2_ref, b2_ref, gfn_ref, bfn_ref, o_ref):
    h = jnp.dot(x_ref[...], w1_ref[...], preferred_element_type=jnp.float32)
    h = _layernorm(h + b1_ref[...], geln_ref[...], beln_ref[...])
    h = jnp.maximum(h, 0.0)
    o = jnp.dot(h, w2_ref[...], preferred_element_type=jnp.float32)
    o_ref[...] = _layernorm(o + b2_ref[...], gfn_ref[...], bfn_ref[...])


def kernel(node_feats, edge_feats, Wq, bq, Wk, bk, Wv, bv, W_topk, b_topk,
           lambda_soft, g_fn, b_fn, W_e1, b_e1, g_eln, b_eln, W_e2, b_e2):
    x = node_feats.reshape(S, HID)
    wt = jnp.pad(W_topk, ((0, 0), (0, 127)))
    Wcat = jnp.concatenate([Wq, Wk, Wv, wt], axis=1)
    bcat = jnp.concatenate([bq, bk, bv, jnp.pad(b_topk, (0, 127))])[None, :]

    y, km = pl.pallas_call(
        _proj_kernel,
        grid=(S // BM,),
        in_specs=[
            pl.BlockSpec((BM, HID), lambda i: (i, 0)),
            pl.BlockSpec((HID, NCOL), lambda i: (0, 0)),
            pl.BlockSpec((1, NCOL), lambda i: (0, 0)),
        ],
        out_specs=[
            pl.BlockSpec((BM, NCOL), lambda i: (i, 0)),
            pl.BlockSpec((BM // CHUNK, HID), lambda i: (i, 0)),
        ],
        out_shape=[
            jax.ShapeDtypeStruct((S, NCOL), jnp.float32),
            jax.ShapeDtypeStruct((C, HID), jnp.float32),
        ],
    )(x, Wcat, bcat)

    kk = pl.pallas_call(
        _kk_kernel,
        grid=(1,),
        in_specs=[pl.BlockSpec((S, 128), lambda i: (0, NQKV // 128))],
        out_specs=pl.BlockSpec(memory_space=pltpu.SMEM),
        out_shape=jax.ShapeDtypeStruct((1, 1), jnp.int32),
    )(y)

    HP = HEADS // 2
    moba = pl.pallas_call(
        _flash_kernel,
        grid=(HP, S // BQ),
        in_specs=[
            pl.BlockSpec(memory_space=pltpu.SMEM),
            pl.BlockSpec((BQ, 2 * DH), lambda p, i: (i, p)),
            pl.BlockSpec((S, 2 * DH), lambda p, i: (0, HP + p)),
            pl.BlockSpec((S, 2 * DH), lambda p, i: (0, 2 * HP + p)),
            pl.BlockSpec((C, 2 * DH), lambda p, i: (0, p)),
        ],
        out_specs=pl.BlockSpec((BQ, 2 * DH), lambda p, i: (i, p)),
        out_shape=jax.ShapeDtypeStruct((S, HID), jnp.float32),
    )(kk, y, y, y, km)

    processed = pl.pallas_call(
        _mlp_kernel,
        grid=(S // BQ,),
        in_specs=[
            pl.BlockSpec((BQ, HID), lambda i: (i, 0)),
            pl.BlockSpec((HID, HID), lambda i: (0, 0)),
            pl.BlockSpec((1, HID), lambda i: (0, 0)),
            pl.BlockSpec((1, HID), lambda i: (0, 0)),
            pl.BlockSpec((1, HID), lambda i: (0, 0)),
            pl.BlockSpec((HID, HID), lambda i: (0, 0)),
            pl.BlockSpec((1, HID), lambda i: (0, 0)),
            pl.BlockSpec((1, HID), lambda i: (0, 0)),
            pl.BlockSpec((1, HID), lambda i: (0, 0)),
        ],
        out_specs=pl.BlockSpec((BQ, HID), lambda i: (i, 0)),
        out_shape=jax.ShapeDtypeStruct((S, HID), jnp.float32),
    )(moba, W_e1, b_e1[None, :], g_eln[None, :], b_eln[None, :],
      W_e2, b_e2[None, :], g_fn[None, :], b_fn[None, :])

    return processed.reshape(1, S, HID), edge_feats


# speedup vs baseline: 1.7286x; 1.7286x over previous
import jax
import jax.numpy as jnp
import numpy as np
from jax.experimental import pallas as pl
from jax.experimental.pallas import tpu as pltpu

S, HID, HEADS = 2048, 1024, 16
CHUNK, TOPK = 64, 4
DH = HID // HEADS
C = S // CHUNK
NQKV = 3 * HID
NCOL = NQKV + 128
BM = 512
BQ = 256
BK = 256
NEG = -1e9


def _proj_kernel(x_ref, w_ref, b_ref, y_ref, km_ref):
    acc = jnp.dot(x_ref[...], w_ref[...], preferred_element_type=jnp.float32)
    acc = acc + b_ref[...]
    y_ref[...] = acc
    r = jax.lax.broadcasted_iota(jnp.int32, (BM // CHUNK, BM), 1)
    c = jax.lax.broadcasted_iota(jnp.int32, (BM // CHUNK, BM), 0)
    m = jnp.where(r // CHUNK == c, 1.0 / CHUNK, 0.0)
    km_ref[...] = jnp.dot(m, acc[:, HID:2 * HID],
                          preferred_element_type=jnp.float32)


def _kk_kernel(ns_ref, kk_ref):
    ns = ns_ref[...]
    col0 = jax.lax.broadcasted_iota(jnp.int32, (S, 128), 1) == 0
    mean = jnp.sum(jnp.where(col0, ns, 0.0)) / S
    cnt = jnp.sum(jnp.where(col0 & (ns > mean), 1, 0))
    cnt = jnp.clip(cnt, 2, S)
    dyn = jnp.floor(jnp.float32(0.8 * float(TOPK))
                    + jnp.float32(0.2) * cnt.astype(jnp.float32))
    kk_ref[0, 0] = jnp.clip(dyn.astype(jnp.int32), 1, C)


def _flash_kernel(kk_ref, q_ref, k_ref, v_ref, km_ref, o_ref):
    i = pl.program_id(1)
    kk = kk_ref[0, 0]
    scale = 1.0 / np.sqrt(DH)
    row = i * BQ + jax.lax.broadcasted_iota(jnp.int32, (BQ, C), 0)
    cid = jax.lax.broadcasted_iota(jnp.int32, (BQ, C), 1)
    qc = row // CHUNK
    qr = i * BQ + jax.lax.broadcasted_iota(jnp.int32, (BQ, BK), 0)
    tE_c = jax.lax.broadcasted_iota(jnp.int32, (C, BK), 0)
    tE_t = jax.lax.broadcasted_iota(jnp.int32, (C, BK), 1) // CHUNK
    kt_loc = jax.lax.broadcasted_iota(jnp.int32, (BQ, BK), 1)

    m0 = jnp.full((BQ, 1), -1e30, jnp.float32)
    l0 = jnp.zeros((BQ, 1), jnp.float32)
    a0 = jnp.zeros((BQ, DH), jnp.float32)

    outs = []
    for t in range(2):
        q = q_ref[:, t * DH:(t + 1) * DH]

        def step(jb, carry, masker):
            m, l, acc = carry
            kb = k_ref[pl.ds(jb * BK, BK), t * DH:(t + 1) * DH]
            vb = v_ref[pl.ds(jb * BK, BK), t * DH:(t + 1) * DH]
            s = jax.lax.dot_general(q, kb, (((1,), (1,)), ((), ())),
                                    preferred_element_type=jnp.float32)
            s = masker(s * scale, jb)
            mn = jnp.maximum(m, jnp.max(s, axis=1, keepdims=True))
            p = jnp.exp(s - mn)
            alpha = jnp.exp(m - mn)
            l2 = l * alpha + jnp.sum(p, axis=1, keepdims=True)
            acc2 = acc * alpha + jnp.dot(p, vb,
                                         preferred_element_type=jnp.float32)
            return mn, l2, acc2

        causal = lambda s, jb: jnp.where(jb * BK + kt_loc <= qr, s, NEG)

        def fast():
            nomask = lambda s, jb: s
            c = jax.lax.fori_loop(0, i, lambda jb, c: step(jb, c, nomask),
                                  (m0, l0, a0))
            return step(i, c, causal)

        def slow():
            km = km_ref[:, t * DH:(t + 1) * DH]
            g = jax.lax.dot_general(q, km, (((1,), (1,)), ((), ())),
                                    preferred_element_type=jnp.float32)
            g = jnp.where(cid > qc, NEG, g)
            g = jnp.where(cid == qc, 1e9, g)
            rank = jnp.zeros((BQ, C), jnp.int32)
            for j in range(C):
                gj = g[:, j:j + 1]
                beat = (gj > g) | ((gj == g) & (j < cid))
                rank = rank + beat.astype(jnp.int32)
            sel = ((rank < kk) & (g > -1e8)).astype(jnp.float32)

            def masker(s, jb):
                e = jnp.where(tE_c == jb * (BK // CHUNK) + tE_t, 1.0, 0.0)
                tok = jnp.dot(sel, e, preferred_element_type=jnp.float32)
                ok = (tok > 0.5) & (jb * BK + kt_loc <= qr)
                return jnp.where(ok, s, NEG)
            c = jax.lax.fori_loop(0, i, lambda jb, c: step(jb, c, masker),
                                  (m0, l0, a0))
            return step(i, c, masker)

        m, l, acc = jax.lax.cond(kk >= C, fast, slow)
        outs.append(acc / l)
    o_ref[...] = jnp.concatenate(outs, axis=1)


def _layernorm(x, g, b):
    mu = jnp.mean(x, axis=1, keepdims=True)
    var = jnp.mean((x - mu) ** 2, axis=1, keepdims=True)
    return (x - mu) / jnp.sqrt(var + 1e-6) * g + b


def _mlp_kernel(x_ref, w1_ref, b1_ref, geln_ref, beln_ref,
                w2_ref, b2_ref, gfn_ref, bfn_ref, o_ref):
    h = jnp.dot(x_ref[...], w1_ref[...], preferred_element_type=jnp.float32)
    h = _layernorm(h + b1_ref[...], geln_ref[...], beln_ref[...])
    h = jnp.maximum(h, 0.0)
    o = jnp.dot(h, w2_ref[...], preferred_element_type=jnp.float32)
    o_ref[...] = _layernorm(o + b2_ref[...], gfn_ref[...], bfn_ref[...])


def kernel(node_feats, edge_feats, Wq, bq, Wk, bk, Wv, bv, W_topk, b_topk,
           lambda_soft, g_fn, b_fn, W_e1, b_e1, g_eln, b_eln, W_e2, b_e2):
    x = node_feats.reshape(S, HID)
    wt = jnp.pad(W_topk, ((0, 0), (0, 127)))
    Wcat = jnp.concatenate([Wq, Wk, Wv, wt], axis=1)
    bcat = jnp.concatenate([bq, bk, bv, jnp.pad(b_topk, (0, 127))])[None, :]

    y, km = pl.pallas_call(
        _proj_kernel,
        grid=(S // BM,),
        in_specs=[
            pl.BlockSpec((BM, HID), lambda i: (i, 0)),
            pl.BlockSpec((HID, NCOL), lambda i: (0, 0)),
            pl.BlockSpec((1, NCOL), lambda i: (0, 0)),
        ],
        out_specs=[
            pl.BlockSpec((BM, NCOL), lambda i: (i, 0)),
            pl.BlockSpec((BM // CHUNK, HID), lambda i: (i, 0)),
        ],
        out_shape=[
            jax.ShapeDtypeStruct((S, NCOL), jnp.float32),
            jax.ShapeDtypeStruct((C, HID), jnp.float32),
        ],
    )(x, Wcat, bcat)

    kk = pl.pallas_call(
        _kk_kernel,
        grid=(1,),
        in_specs=[pl.BlockSpec((S, 128), lambda i: (0, NQKV // 128))],
        out_specs=pl.BlockSpec(memory_space=pltpu.SMEM),
        out_shape=jax.ShapeDtypeStruct((1, 1), jnp.int32),
    )(y)

    HP = HEADS // 2
    moba = pl.pallas_call(
        _flash_kernel,
        grid=(HP, S // BQ),
        in_specs=[
            pl.BlockSpec(memory_space=pltpu.SMEM),
            pl.BlockSpec((BQ, 2 * DH), lambda p, i: (i, p)),
            pl.BlockSpec((S, 2 * DH), lambda p, i: (0, HP + p)),
            pl.BlockSpec((S, 2 * DH), lambda p, i: (0, 2 * HP + p)),
            pl.BlockSpec((C, 2 * DH), lambda p, i: (0, p)),
        ],
        out_specs=pl.BlockSpec((BQ, 2 * DH), lambda p, i: (i, p)),
        out_shape=jax.ShapeDtypeStruct((S, HID), jnp.float32),
    )(kk, y, y, y, km)

    processed = pl.pallas_call(
        _mlp_kernel,
        grid=(S // BQ,),
        in_specs=[
            pl.BlockSpec((BQ, HID), lambda i: (i, 0)),
            pl.BlockSpec((HID, HID), lambda i: (0, 0)),
            pl.BlockSpec((1, HID), lambda i: (0, 0)),
            pl.BlockSpec((1, HID), lambda i: (0, 0)),
            pl.BlockSpec((1, HID), lambda i: (0, 0)),
            pl.BlockSpec((HID, HID), lambda i: (0, 0)),
            pl.BlockSpec((1, HID), lambda i: (0, 0)),
            pl.BlockSpec((1, HID), lambda i: (0, 0)),
            pl.BlockSpec((1, HID), lambda i: (0, 0)),
        ],
        out_specs=pl.BlockSpec((BQ, HID), lambda i: (i, 0)),
        out_shape=jax.ShapeDtypeStruct((S, HID), jnp.float32),
    )(moba, W_e1, b_e1[None, :], g_eln[None, :], b_eln[None, :],
      W_e2, b_e2[None, :], g_fn[None, :], b_fn[None, :])

    return processed.reshape(1, S, HID), edge_feats


# speedup vs baseline: 2.3547x; 1.3622x over previous
import jax
import jax.numpy as jnp
import numpy as np
from jax.experimental import pallas as pl
from jax.experimental.pallas import tpu as pltpu

S, HID, HEADS = 2048, 1024, 16
CHUNK, TOPK = 64, 4
DH = HID // HEADS
C = S // CHUNK
NQKV = 3 * HID
NCOL = NQKV + 128
BM = 512
BQ = 256
BK = 512
NEG = -1e9


def _proj_kernel(x_ref, w_ref, b_ref, qkv_ref, q32_ref, ns_ref, km_ref):
    acc = jnp.dot(x_ref[...], w_ref[...], preferred_element_type=jnp.float32)
    acc = acc + b_ref[...]
    qkv_ref[...] = acc[:, :NQKV].astype(jnp.bfloat16)
    q32_ref[...] = acc[:, :HID]
    ns_ref[...] = acc[:, NQKV:]
    r = jax.lax.broadcasted_iota(jnp.int32, (BM // CHUNK, BM), 1)
    c = jax.lax.broadcasted_iota(jnp.int32, (BM // CHUNK, BM), 0)
    m = jnp.where(r // CHUNK == c, 1.0 / CHUNK, 0.0)
    km_ref[...] = jnp.dot(m, acc[:, HID:2 * HID],
                          preferred_element_type=jnp.float32)


def _kk_kernel(ns_ref, kk_ref):
    ns = ns_ref[...]
    col0 = jax.lax.broadcasted_iota(jnp.int32, (S, 128), 1) == 0
    mean = jnp.sum(jnp.where(col0, ns, 0.0)) / S
    cnt = jnp.sum(jnp.where(col0 & (ns > mean), 1, 0))
    cnt = jnp.clip(cnt, 2, S)
    dyn = jnp.floor(jnp.float32(0.8 * float(TOPK))
                    + jnp.float32(0.2) * cnt.astype(jnp.float32))
    kk_ref[0, 0] = jnp.clip(dyn.astype(jnp.int32), 1, C)


def _flash_kernel(kk_ref, q_ref, k_ref, v_ref, q32_ref, km_ref, o_ref):
    i = pl.program_id(1)
    kk = kk_ref[0, 0]
    scale = np.float32(1.0 / np.sqrt(DH))
    nb = i // 2 + 1
    row = i * BQ + jax.lax.broadcasted_iota(jnp.int32, (BQ, C), 0)
    cid = jax.lax.broadcasted_iota(jnp.int32, (BQ, C), 1)
    qc = row // CHUNK
    qr = i * BQ + jax.lax.broadcasted_iota(jnp.int32, (BQ, BK), 0)
    tE_c = jax.lax.broadcasted_iota(jnp.int32, (C, BK), 0)
    tE_t = jax.lax.broadcasted_iota(jnp.int32, (C, BK), 1) // CHUNK
    kt_loc = jax.lax.broadcasted_iota(jnp.int32, (BQ, BK), 1)

    m0 = jnp.full((BQ, 1), -1e30, jnp.float32)
    l0 = jnp.zeros((BQ, 1), jnp.float32)
    a0 = jnp.zeros((BQ, DH), jnp.float32)

    outs = []
    for t in range(2):
        q = q_ref[:, t * DH:(t + 1) * DH]

        def step(jb, carry, masker):
            m, l, acc = carry
            kb = k_ref[pl.ds(jb * BK, BK), t * DH:(t + 1) * DH]
            vb = v_ref[pl.ds(jb * BK, BK), t * DH:(t + 1) * DH]
            s = jax.lax.dot_general(q, kb, (((1,), (1,)), ((), ())),
                                    preferred_element_type=jnp.float32)
            s = masker(s * scale, jb)
            mn = jnp.maximum(m, jnp.max(s, axis=1, keepdims=True))
            p = jnp.exp(s - mn)
            alpha = jnp.exp(m - mn)
            l2 = l * alpha + jnp.sum(p, axis=1, keepdims=True)
            acc2 = acc * alpha + jnp.dot(p.astype(jnp.bfloat16), vb,
                                         preferred_element_type=jnp.float32)
            return mn, l2, acc2

        causal = lambda s, jb: jnp.where(jb * BK + kt_loc <= qr, s, NEG)

        def fast():
            nomask = lambda s, jb: s
            c = jax.lax.fori_loop(0, nb - 1, lambda jb, c: step(jb, c, nomask),
                                  (m0, l0, a0))
            return step(nb - 1, c, causal)

        def slow():
            km = km_ref[:, t * DH:(t + 1) * DH]
            q32 = q32_ref[:, t * DH:(t + 1) * DH]
            g = jax.lax.dot_general(q32, km, (((1,), (1,)), ((), ())),
                                    preferred_element_type=jnp.float32)
            g = jnp.where(cid > qc, NEG, g)
            g = jnp.where(cid == qc, 1e9, g)
            rank = jnp.zeros((BQ, C), jnp.int32)
            for j in range(C):
                gj = g[:, j:j + 1]
                beat = (gj > g) | ((gj == g) & (j < cid))
                rank = rank + beat.astype(jnp.int32)
            sel = ((rank < kk) & (g > -1e8)).astype(jnp.bfloat16)

            def masker(s, jb):
                e = jnp.where(tE_c == jb * (BK // CHUNK) + tE_t, 1.0,
                              0.0).astype(jnp.bfloat16)
                tok = jnp.dot(sel, e, preferred_element_type=jnp.float32)
                ok = (tok > 0.5) & (jb * BK + kt_loc <= qr)
                return jnp.where(ok, s, NEG)
            c = jax.lax.fori_loop(0, nb - 1, lambda jb, c: step(jb, c, masker),
                                  (m0, l0, a0))
            return step(nb - 1, c, masker)

        m, l, acc = jax.lax.cond(kk >= C, fast, slow)
        outs.append(acc / l)
    o_ref[...] = jnp.concatenate(outs, axis=1)


def _layernorm(x, g, b):
    mu = jnp.mean(x, axis=1, keepdims=True)
    var = jnp.mean((x - mu) ** 2, axis=1, keepdims=True)
    return (x - mu) / jnp.sqrt(var + 1e-6) * g + b


def _mlp_kernel(x_ref, w1_ref, b1_ref, geln_ref, beln_ref,
                w2_ref, b2_ref, gfn_ref, bfn_ref, o_ref):
    h = jnp.dot(x_ref[...], w1_ref[...], preferred_element_type=jnp.float32)
    h = _layernorm(h + b1_ref[...], geln_ref[...], beln_ref[...])
    h = jnp.maximum(h, 0.0)
    o = jnp.dot(h, w2_ref[...], preferred_element_type=jnp.float32)
    o_ref[...] = _layernorm(o + b2_ref[...], gfn_ref[...], bfn_ref[...])


def kernel(node_feats, edge_feats, Wq, bq, Wk, bk, Wv, bv, W_topk, b_topk,
           lambda_soft, g_fn, b_fn, W_e1, b_e1, g_eln, b_eln, W_e2, b_e2):
    x = node_feats.reshape(S, HID)
    wt = jnp.pad(W_topk, ((0, 0), (0, 127)))
    Wcat = jnp.concatenate([Wq, Wk, Wv, wt], axis=1)
    bcat = jnp.concatenate([bq, bk, bv, jnp.pad(b_topk, (0, 127))])[None, :]

    qkv, q32, ns, km = pl.pallas_call(
        _proj_kernel,
        grid=(S // BM,),
        in_specs=[
            pl.BlockSpec((BM, HID), lambda i: (i, 0)),
            pl.BlockSpec((HID, NCOL), lambda i: (0, 0)),
            pl.BlockSpec((1, NCOL), lambda i: (0, 0)),
        ],
        out_specs=[
            pl.BlockSpec((BM, NQKV), lambda i: (i, 0)),
            pl.BlockSpec((BM, HID), lambda i: (i, 0)),
            pl.BlockSpec((BM, 128), lambda i: (i, 0)),
            pl.BlockSpec((BM // CHUNK, HID), lambda i: (i, 0)),
        ],
        out_shape=[
            jax.ShapeDtypeStruct((S, NQKV), jnp.bfloat16),
            jax.ShapeDtypeStruct((S, HID), jnp.float32),
            jax.ShapeDtypeStruct((S, 128), jnp.float32),
            jax.ShapeDtypeStruct((C, HID), jnp.float32),
        ],
    )(x, Wcat, bcat)

    kk = pl.pallas_call(
        _kk_kernel,
        grid=(1,),
        in_specs=[pl.BlockSpec((S, 128), lambda i: (0, 0))],
        out_specs=pl.BlockSpec(memory_space=pltpu.SMEM),
        out_shape=jax.ShapeDtypeStruct((1, 1), jnp.int32),
    )(ns)

    HP = HEADS // 2
    moba = pl.pallas_call(
        _flash_kernel,
        grid=(HP, S // BQ),
        in_specs=[
            pl.BlockSpec(memory_space=pltpu.SMEM),
            pl.BlockSpec((BQ, 2 * DH), lambda p, i: (i, p)),
            pl.BlockSpec((S, 2 * DH), lambda p, i: (0, HP + p)),
            pl.BlockSpec((S, 2 * DH), lambda p, i: (0, 2 * HP + p)),
            pl.BlockSpec((BQ, 2 * DH), lambda p, i: (i, p)),
            pl.BlockSpec((C, 2 * DH), lambda p, i: (0, p)),
        ],
        out_specs=pl.BlockSpec((BQ, 2 * DH), lambda p, i: (i, p)),
        out_shape=jax.ShapeDtypeStruct((S, HID), jnp.float32),
    )(kk, qkv, qkv, qkv, q32, km)

    processed = pl.pallas_call(
        _mlp_kernel,
        grid=(S // BQ,),
        in_specs=[
            pl.BlockSpec((BQ, HID), lambda i: (i, 0)),
            pl.BlockSpec((HID, HID), lambda i: (0, 0)),
            pl.BlockSpec((1, HID), lambda i: (0, 0)),
            pl.BlockSpec((1, HID), lambda i: (0, 0)),
            pl.BlockSpec((1, HID), lambda i: (0, 0)),
            pl.BlockSpec((HID, HID), lambda i: (0, 0)),
            pl.BlockSpec((1, HID), lambda i: (0, 0)),
            pl.BlockSpec((1, HID), lambda i: (0, 0)),
            pl.BlockSpec((1, HID), lambda i: (0, 0)),
        ],
        out_specs=pl.BlockSpec((BQ, HID), lambda i: (i, 0)),
        out_shape=jax.ShapeDtypeStruct((S, HID), jnp.float32),
    )(moba, W_e1, b_e1[None, :], g_eln[None, :], b_eln[None, :],
      W_e2, b_e2[None, :], g_fn[None, :], b_fn[None, :])

    return processed.reshape(1, S, HID), edge_feats


# speedup vs baseline: 2.6057x; 1.1066x over previous
import jax
import jax.numpy as jnp
import numpy as np
from jax.experimental import pallas as pl
from jax.experimental.pallas import tpu as pltpu

S, HID, HEADS = 2048, 1024, 16
CHUNK, TOPK = 64, 4
DH = HID // HEADS
C = S // CHUNK
NQKV = 3 * HID
NCOL = NQKV + 128
BM = 512
BQ = 256
BK = 512
NEG = -1e9


def _proj_kernel(x_ref, w_ref, b_ref, qkv_ref, q32_ref, ns_ref, km_ref):
    acc = jnp.dot(x_ref[...], w_ref[...], preferred_element_type=jnp.float32)
    acc = acc + b_ref[...]
    qkv_ref[...] = acc[:, :NQKV].astype(jnp.bfloat16)
    q32_ref[...] = acc[:, :HID]
    ns_ref[...] = acc[:, NQKV:]
    r = jax.lax.broadcasted_iota(jnp.int32, (BM // CHUNK, BM), 1)
    c = jax.lax.broadcasted_iota(jnp.int32, (BM // CHUNK, BM), 0)
    m = jnp.where(r // CHUNK == c, 1.0 / CHUNK, 0.0)
    km_ref[...] = jnp.dot(m, acc[:, HID:2 * HID],
                          preferred_element_type=jnp.float32)


def _kk_kernel(ns_ref, kk_ref):
    ns = ns_ref[...]
    col0 = jax.lax.broadcasted_iota(jnp.int32, (S, 128), 1) == 0
    mean = jnp.sum(jnp.where(col0, ns, 0.0)) / S
    cnt = jnp.sum(jnp.where(col0 & (ns > mean), 1, 0))
    cnt = jnp.clip(cnt, 2, S)
    dyn = jnp.floor(jnp.float32(0.8 * float(TOPK))
                    + jnp.float32(0.2) * cnt.astype(jnp.float32))
    kk_ref[0, 0] = jnp.clip(dyn.astype(jnp.int32), 1, C)


def _flash_kernel(kk_ref, q_ref, k_ref, v_ref, q32_ref, km_ref, o_ref):
    i = pl.program_id(1)
    kk = kk_ref[0, 0]
    scale = np.float32(1.0 / np.sqrt(DH))
    nb = i // 2 + 1
    row = i * BQ + jax.lax.broadcasted_iota(jnp.int32, (BQ, C), 0)
    cid = jax.lax.broadcasted_iota(jnp.int32, (BQ, C), 1)
    qc = row // CHUNK
    qr = i * BQ + jax.lax.broadcasted_iota(jnp.int32, (BQ, BK), 0)
    tE_c = jax.lax.broadcasted_iota(jnp.int32, (C, BK), 0)
    tE_t = jax.lax.broadcasted_iota(jnp.int32, (C, BK), 1) // CHUNK
    kt_loc = jax.lax.broadcasted_iota(jnp.int32, (BQ, BK), 1)

    m0 = jnp.full((BQ, 1), -1e30, jnp.float32)
    l0 = jnp.zeros((BQ, 1), jnp.float32)
    a0 = jnp.zeros((BQ, DH), jnp.float32)
    init = (m0, l0, a0, m0, l0, a0)
    qs = [q_ref[:, t * DH:(t + 1) * DH] for t in range(2)]

    def step(jb, carry, maskers):
        kb2 = k_ref[pl.ds(jb * BK, BK), :]
        vb2 = v_ref[pl.ds(jb * BK, BK), :]
        out = []
        for t in range(2):
            m, l, acc = carry[3 * t:3 * t + 3]
            kb = kb2[:, t * DH:(t + 1) * DH]
            vb = vb2[:, t * DH:(t + 1) * DH]
            s = jax.lax.dot_general(qs[t], kb, (((1,), (1,)), ((), ())),
                                    preferred_element_type=jnp.float32)
            s = maskers[t](s * scale, jb)
            mn = jnp.maximum(m, jnp.max(s, axis=1, keepdims=True))
            p = jnp.exp(s - mn)
            alpha = jnp.exp(m - mn)
            l2 = l * alpha + jnp.sum(p, axis=1, keepdims=True)
            acc2 = acc * alpha + jnp.dot(p.astype(jnp.bfloat16), vb,
                                         preferred_element_type=jnp.float32)
            out += [mn, l2, acc2]
        return tuple(out)

    causal = lambda s, jb: jnp.where(jb * BK + kt_loc <= qr, s, NEG)

    def fast():
        nomask = lambda s, jb: s
        c = jax.lax.fori_loop(
            0, nb - 1, lambda jb, c: step(jb, c, (nomask, nomask)), init)
        return step(nb - 1, c, (causal, causal))

    def slow():
        maskers = []
        for t in range(2):
            km = km_ref[:, t * DH:(t + 1) * DH]
            q32 = q32_ref[:, t * DH:(t + 1) * DH]
            g = jax.lax.dot_general(q32, km, (((1,), (1,)), ((), ())),
                                    preferred_element_type=jnp.float32)
            g = jnp.where(cid > qc, NEG, g)
            g = jnp.where(cid == qc, 1e9, g)
            rank = jnp.zeros((BQ, C), jnp.int32)
            for j in range(C):
                gj = g[:, j:j + 1]
                beat = (gj > g) | ((gj == g) & (j < cid))
                rank = rank + beat.astype(jnp.int32)
            sel = ((rank < kk) & (g > -1e8)).astype(jnp.bfloat16)

            def masker(s, jb, sel=sel):
                e = jnp.where(tE_c == jb * (BK // CHUNK) + tE_t, 1.0,
                              0.0).astype(jnp.bfloat16)
                tok = jnp.dot(sel, e, preferred_element_type=jnp.float32)
                ok = (tok > 0.5) & (jb * BK + kt_loc <= qr)
                return jnp.where(ok, s, NEG)
            maskers.append(masker)
        maskers = tuple(maskers)
        c = jax.lax.fori_loop(
            0, nb - 1, lambda jb, c: step(jb, c, maskers), init)
        return step(nb - 1, c, maskers)

    r = jax.lax.cond(kk >= C, fast, slow)
    o_ref[...] = jnp.concatenate([r[2] / r[1], r[5] / r[4]], axis=1)


def _layernorm(x, g, b):
    mu = jnp.mean(x, axis=1, keepdims=True)
    var = jnp.mean((x - mu) ** 2, axis=1, keepdims=True)
    return (x - mu) / jnp.sqrt(var + 1e-6) * g + b


def _mlp_kernel(x_ref, w1_ref, b1_ref, geln_ref, beln_ref,
                w2_ref, b2_ref, gfn_ref, bfn_ref, o_ref):
    h = jnp.dot(x_ref[...], w1_ref[...], preferred_element_type=jnp.float32)
    h = _layernorm(h + b1_ref[...], geln_ref[...], beln_ref[...])
    h = jnp.maximum(h, 0.0)
    o = jnp.dot(h, w2_ref[...], preferred_element_type=jnp.float32)
    o_ref[...] = _layernorm(o + b2_ref[...], gfn_ref[...], bfn_ref[...])


def kernel(node_feats, edge_feats, Wq, bq, Wk, bk, Wv, bv, W_topk, b_topk,
           lambda_soft, g_fn, b_fn, W_e1, b_e1, g_eln, b_eln, W_e2, b_e2):
    x = node_feats.reshape(S, HID)
    wt = jnp.pad(W_topk, ((0, 0), (0, 127)))
    Wcat = jnp.concatenate([Wq, Wk, Wv, wt], axis=1)
    bcat = jnp.concatenate([bq, bk, bv, jnp.pad(b_topk, (0, 127))])[None, :]

    qkv, q32, ns, km = pl.pallas_call(
        _proj_kernel,
        grid=(S // BM,),
        in_specs=[
            pl.BlockSpec((BM, HID), lambda i: (i, 0)),
            pl.BlockSpec((HID, NCOL), lambda i: (0, 0)),
            pl.BlockSpec((1, NCOL), lambda i: (0, 0)),
        ],
        out_specs=[
            pl.BlockSpec((BM, NQKV), lambda i: (i, 0)),
            pl.BlockSpec((BM, HID), lambda i: (i, 0)),
            pl.BlockSpec((BM, 128), lambda i: (i, 0)),
            pl.BlockSpec((BM // CHUNK, HID), lambda i: (i, 0)),
        ],
        out_shape=[
            jax.ShapeDtypeStruct((S, NQKV), jnp.bfloat16),
            jax.ShapeDtypeStruct((S, HID), jnp.float32),
            jax.ShapeDtypeStruct((S, 128), jnp.float32),
            jax.ShapeDtypeStruct((C, HID), jnp.float32),
        ],
    )(x, Wcat, bcat)

    kk = pl.pallas_call(
        _kk_kernel,
        grid=(1,),
        in_specs=[pl.BlockSpec((S, 128), lambda i: (0, 0))],
        out_specs=pl.BlockSpec(memory_space=pltpu.SMEM),
        out_shape=jax.ShapeDtypeStruct((1, 1), jnp.int32),
    )(ns)

    HP = HEADS // 2
    moba = pl.pallas_call(
        _flash_kernel,
        grid=(HP, S // BQ),
        in_specs=[
            pl.BlockSpec(memory_space=pltpu.SMEM),
            pl.BlockSpec((BQ, 2 * DH), lambda p, i: (i, p)),
            pl.BlockSpec((S, 2 * DH), lambda p, i: (0, HP + p)),
            pl.BlockSpec((S, 2 * DH), lambda p, i: (0, 2 * HP + p)),
            pl.BlockSpec((BQ, 2 * DH), lambda p, i: (i, p)),
            pl.BlockSpec((C, 2 * DH), lambda p, i: (0, p)),
        ],
        out_specs=pl.BlockSpec((BQ, 2 * DH), lambda p, i: (i, p)),
        out_shape=jax.ShapeDtypeStruct((S, HID), jnp.float32),
    )(kk, qkv, qkv, qkv, q32, km)

    processed = pl.pallas_call(
        _mlp_kernel,
        grid=(S // BQ,),
        in_specs=[
            pl.BlockSpec((BQ, HID), lambda i: (i, 0)),
            pl.BlockSpec((HID, HID), lambda i: (0, 0)),
            pl.BlockSpec((1, HID), lambda i: (0, 0)),
            pl.BlockSpec((1, HID), lambda i: (0, 0)),
            pl.BlockSpec((1, HID), lambda i: (0, 0)),
            pl.BlockSpec((HID, HID), lambda i: (0, 0)),
            pl.BlockSpec((1, HID), lambda i: (0, 0)),
            pl.BlockSpec((1, HID), lambda i: (0, 0)),
            pl.BlockSpec((1, HID), lambda i: (0, 0)),
        ],
        out_specs=pl.BlockSpec((BQ, HID), lambda i: (i, 0)),
        out_shape=jax.ShapeDtypeStruct((S, HID), jnp.float32),
    )(moba, W_e1, b_e1[None, :], g_eln[None, :], b_eln[None, :],
      W_e2, b_e2[None, :], g_fn[None, :], b_fn[None, :])

    return processed.reshape(1, S, HID), edge_feats


# speedup vs baseline: 3.6639x; 1.4061x over previous
import jax
import jax.numpy as jnp
import numpy as np
from jax.experimental import pallas as pl
from jax.experimental.pallas import tpu as pltpu
from jax.experimental.pallas import tpu_sc as plsc

S, HID, HEADS = 2048, 1024, 16
CHUNK, TOPK = 64, 4
DH = HID // HEADS
C = S // CHUNK
NQKV = 3 * HID
BM = 512
BQ = 512
BK = 1024
NEG = -1e9


NCOL = NQKV + 128


def _proj_kernel(x_ref, w_ref, b_ref, qkv_ref, q32_ref, ns_ref, km_ref):
    acc = jnp.dot(x_ref[...], w_ref[...], preferred_element_type=jnp.float32)
    acc = acc + b_ref[...]
    qkv_ref[...] = acc[:, :NQKV].astype(jnp.bfloat16)
    q32_ref[...] = acc[:, :HID]
    ns_ref[...] = acc[:, NQKV:]
    r = jax.lax.broadcasted_iota(jnp.int32, (BM // CHUNK, BM), 1)
    c = jax.lax.broadcasted_iota(jnp.int32, (BM // CHUNK, BM), 0)
    m = jnp.where(r // CHUNK == c, 1.0 / CHUNK, 0.0)
    km_ref[...] = jnp.dot(m, acc[:, HID:2 * HID],
                          preferred_element_type=jnp.float32)


def _kk_sc_kernel(ns_hbm, kk_hbm, nsv, kkv):
    wid = jax.lax.axis_index("s") * 2 + jax.lax.axis_index("c")
    pltpu.sync_copy(ns_hbm, nsv)
    acc = jnp.zeros((16,), jnp.float32)
    for c in range(S // 16):
        acc = acc + nsv[c * 16:(c + 1) * 16]
    total = acc[0]
    for j in range(1, 16):
        total = total + acc[j]
    mean = total * jnp.float32(1.0 / S)
    cnt = jnp.zeros((16,), jnp.int32)
    for c in range(S // 16):
        cnt = cnt + jnp.where(nsv[c * 16:(c + 1) * 16] > mean, 1, 0)
    cnts = cnt[0]
    for j in range(1, 16):
        cnts = cnts + cnt[j]
    cntc = jnp.clip(cnts, 2, S)
    dyn = (jnp.float32(0.8 * float(TOPK))
           + jnp.float32(0.2) * cntc.astype(jnp.float32))
    kkv[...] = (jnp.clip(dyn.astype(jnp.int32), 1, C)
                + jnp.zeros((16,), jnp.int32))

    @pl.when(wid == 0)
    def _():
        pltpu.sync_copy(kkv, kk_hbm)


def _routing_kk(ns_col):
    mesh = plsc.VectorSubcoreMesh(core_axis_name="c", subcore_axis_name="s")
    return pl.kernel(
        _kk_sc_kernel,
        out_type=jax.ShapeDtypeStruct((16,), jnp.int32),
        scratch_types=[
            pltpu.VMEM((S,), jnp.float32),
            pltpu.VMEM((16,), jnp.int32),
        ],
        mesh=mesh,
    )(ns_col)


def _attend(i, q_ref, k_ref, v_ref, masker, mask_all=False):
    nb = ((i + 1) * BQ + BK - 1) // BK
    m0 = jnp.full((BQ, 1), -1e30, jnp.float32)
    a0 = jnp.zeros((BQ, 2 * DH), jnp.float32)
    init = (m0, a0, m0, a0)
    ones = jnp.ones((BK, DH), jnp.bfloat16)
    qs = [q_ref[:, t * DH:(t + 1) * DH] * jnp.bfloat16(0.125)
          for t in range(2)]

    def make_step(masked):
        def step(jb, carry):
            kb2 = k_ref[pl.ds(jb * BK, BK), :]
            vb2 = v_ref[pl.ds(jb * BK, BK), :]
            out = []
            for t in range(2):
                m, acc = carry[2 * t:2 * t + 2]
                kb = kb2[:, t * DH:(t + 1) * DH]
                vaug = jnp.concatenate(
                    [vb2[:, t * DH:(t + 1) * DH], ones], axis=1)
                s = jax.lax.dot_general(qs[t], kb, (((1,), (1,)), ((), ())),
                                        preferred_element_type=jnp.float32)
                if masked:
                    s = masker(s, jb, t)
                mn = jnp.maximum(m, jnp.max(s, axis=1, keepdims=True))
                p = jnp.exp(s - mn)
                alpha = jnp.exp(m - mn)
                acc2 = acc * alpha + jnp.dot(p.astype(jnp.bfloat16), vaug,
                                             preferred_element_type=jnp.float32)
                out += [mn, acc2]
            return tuple(out)
        return step

    c = jax.lax.fori_loop(0, nb - 1, make_step(mask_all), init)
    r = make_step(True)(nb - 1, c)
    return jnp.concatenate([r[1][:, :DH] / r[1][:, DH:DH + 1],
                            r[3][:, :DH] / r[3][:, DH:DH + 1]], axis=1)


def _flash_fast_kernel(q_ref, k_ref, v_ref, o_ref):
    i = pl.program_id(1)
    qr = i * BQ + jax.lax.broadcasted_iota(jnp.int32, (BQ, BK), 0)
    kt_loc = jax.lax.broadcasted_iota(jnp.int32, (BQ, BK), 1)
    causal = lambda s, jb, t: jnp.where(jb * BK + kt_loc <= qr, s, NEG)
    o_ref[...] = _attend(i, q_ref, k_ref, v_ref, causal)


def _flash_slow_kernel(kk_ref, q_ref, k_ref, v_ref, q32_ref, km_ref, o_ref):
    i = pl.program_id(1)
    kk = kk_ref[0, 0]
    row = i * BQ + jax.lax.broadcasted_iota(jnp.int32, (BQ, C), 0)
    cid = jax.lax.broadcasted_iota(jnp.int32, (BQ, C), 1)
    qc = row // CHUNK
    qr = i * BQ + jax.lax.broadcasted_iota(jnp.int32, (BQ, BK), 0)
    tE_c = jax.lax.broadcasted_iota(jnp.int32, (C, BK), 0)
    tE_t = jax.lax.broadcasted_iota(jnp.int32, (C, BK), 1) // CHUNK
    kt_loc = jax.lax.broadcasted_iota(jnp.int32, (BQ, BK), 1)

    sels = []
    for t in range(2):
        km = km_ref[:, t * DH:(t + 1) * DH]
        q32 = q32_ref[:, t * DH:(t + 1) * DH]
        g = jax.lax.dot_general(q32, km, (((1,), (1,)), ((), ())),
                                preferred_element_type=jnp.float32)
        g = jnp.where(cid > qc, NEG, g)
        g = jnp.where(cid == qc, 1e9, g)
        rank = jnp.zeros((BQ, C), jnp.int32)
        for j in range(C):
            gj = g[:, j:j + 1]
            beat = (gj > g) | ((gj == g) & (j < cid))
            rank = rank + beat.astype(jnp.int32)
        sels.append(((rank < kk) & (g > -1e8)).astype(jnp.bfloat16))

    def masker(s, jb, t):
        e = jnp.where(tE_c == jb * (BK // CHUNK) + tE_t, 1.0,
                      0.0).astype(jnp.bfloat16)
        tok = jnp.dot(sels[t], e, preferred_element_type=jnp.float32)
        ok = (tok > 0.5) & (jb * BK + kt_loc <= qr)
        return jnp.where(ok, s, NEG)

    o_ref[...] = _attend(i, q_ref, k_ref, v_ref, masker, mask_all=True)


def _layernorm(x, g, b):
    mu = jnp.mean(x, axis=1, keepdims=True)
    var = jnp.mean((x - mu) ** 2, axis=1, keepdims=True)
    return (x - mu) / jnp.sqrt(var + 1e-6) * g + b


def _mlp_kernel(x_ref, w1_ref, b1_ref, geln_ref, beln_ref,
                w2_ref, b2_ref, gfn_ref, bfn_ref, o_ref):
    h = jnp.dot(x_ref[...], w1_ref[...], preferred_element_type=jnp.float32)
    h = _layernorm(h + b1_ref[...], geln_ref[...], beln_ref[...])
    h = jnp.maximum(h, 0.0)
    o = jnp.dot(h, w2_ref[...], preferred_element_type=jnp.float32)
    o_ref[...] = _layernorm(o + b2_ref[...], gfn_ref[...], bfn_ref[...])


def kernel(node_feats, edge_feats, Wq, bq, Wk, bk, Wv, bv, W_topk, b_topk,
           lambda_soft, g_fn, b_fn, W_e1, b_e1, g_eln, b_eln, W_e2, b_e2):
    x = node_feats.reshape(S, HID)
    wt = jnp.pad(W_topk, ((0, 0), (0, 127)))
    Wcat = jnp.concatenate([Wq, Wk, Wv, wt], axis=1)
    bcat = jnp.concatenate([bq, bk, bv, jnp.pad(b_topk, (0, 127))])[None, :]

    qkv, q32, ns, km = pl.pallas_call(
        _proj_kernel,
        grid=(S // BM,),
        in_specs=[
            pl.BlockSpec((BM, HID), lambda i: (i, 0)),
            pl.BlockSpec((HID, NCOL), lambda i: (0, 0)),
            pl.BlockSpec((1, NCOL), lambda i: (0, 0)),
        ],
        out_specs=[
            pl.BlockSpec((BM, NQKV), lambda i: (i, 0)),
            pl.BlockSpec((BM, HID), lambda i: (i, 0)),
            pl.BlockSpec((BM, 128), lambda i: (i, 0)),
            pl.BlockSpec((BM // CHUNK, HID), lambda i: (i, 0)),
        ],
        out_shape=[
            jax.ShapeDtypeStruct((S, NQKV), jnp.bfloat16),
            jax.ShapeDtypeStruct((S, HID), jnp.float32),
            jax.ShapeDtypeStruct((S, 128), jnp.float32),
            jax.ShapeDtypeStruct((C, HID), jnp.float32),
        ],
    )(x, Wcat, bcat)

    kk = _routing_kk(ns[:, 0])[:1].reshape(1, 1)

    HP = HEADS // 2
    grid = (HP, S // BQ)
    qspec = pl.BlockSpec((BQ, 2 * DH), lambda p, i: (i, p))
    kspec = pl.BlockSpec((S, 2 * DH), lambda p, i: (0, HP + p))
    vspec = pl.BlockSpec((S, 2 * DH), lambda p, i: (0, 2 * HP + p))
    ospec = pl.BlockSpec((BQ, 2 * DH), lambda p, i: (i, p))
    oshape = jax.ShapeDtypeStruct((S, HID), jnp.float32)

    def call_fast():
        return pl.pallas_call(
            _flash_fast_kernel, grid=grid,
            in_specs=[qspec, kspec, vspec],
            out_specs=ospec, out_shape=oshape,
        )(qkv, qkv, qkv)

    def call_slow():
        return pl.pallas_call(
            _flash_slow_kernel, grid=grid,
            in_specs=[
                pl.BlockSpec(memory_space=pltpu.SMEM),
                qspec, kspec, vspec,
                pl.BlockSpec((BQ, 2 * DH), lambda p, i: (i, p)),
                pl.BlockSpec((C, 2 * DH), lambda p, i: (0, p)),
            ],
            out_specs=ospec, out_shape=oshape,
        )(kk, qkv, qkv, qkv, q32, km)

    moba_fast = call_fast()
    moba = jax.lax.cond(kk[0, 0] >= C, lambda: moba_fast, call_slow)

    processed = pl.pallas_call(
        _mlp_kernel,
        grid=(S // BQ,),
        in_specs=[
            pl.BlockSpec((BQ, HID), lambda i: (i, 0)),
            pl.BlockSpec((HID, HID), lambda i: (0, 0)),
            pl.BlockSpec((1, HID), lambda i: (0, 0)),
            pl.BlockSpec((1, HID), lambda i: (0, 0)),
            pl.BlockSpec((1, HID), lambda i: (0, 0)),
            pl.BlockSpec((HID, HID), lambda i: (0, 0)),
            pl.BlockSpec((1, HID), lambda i: (0, 0)),
            pl.BlockSpec((1, HID), lambda i: (0, 0)),
            pl.BlockSpec((1, HID), lambda i: (0, 0)),
        ],
        out_specs=pl.BlockSpec((BQ, HID), lambda i: (i, 0)),
        out_shape=jax.ShapeDtypeStruct((S, HID), jnp.float32),
    )(moba, W_e1, b_e1[None, :], g_eln[None, :], b_eln[None, :],
      W_e2, b_e2[None, :], g_fn[None, :], b_fn[None, :])

    return processed.reshape(1, S, HID), edge_feats
